# baseline (speedup 1.0000x reference)
import jax
import jax.numpy as jnp
from jax.experimental import pallas as pl
from jax.experimental.pallas import tpu as pltpu

Q = 8
K = 1024
D = 256
BETA = 0.25


NCHAIN = 4


def _rvq_body(x_ref, cb_ref, cbs_ref, qout_ref, idx_ref, loss_ref):
    r = x_ref.shape[0] // NCHAIN
    iota_k = jax.lax.broadcasted_iota(jnp.int32, (r, K), 1)
    res = [x_ref[h * r:(h + 1) * r] for h in range(NCHAIN)]
    qacc = [jnp.zeros_like(res[0]) for _ in range(NCHAIN)]
    idx_rows = [[] for _ in range(NCHAIN)]
    loss_rows = []
    for q in range(Q):
        cb = cb_ref[q]
        cnorm = jnp.sum(cb * cb, axis=1)
        loss_q = []
        for h in range(NCHAIN):
            mm = jax.lax.dot_general(
                res[h], cb, (((1,), (1,)), ((), ())),
                preferred_element_type=jnp.float32)
            rown = jnp.sum(res[h] * res[h], axis=1, keepdims=True)
            dist = rown - 2.0 * mm + cnorm[None, :]
            m = jnp.min(dist, axis=1, keepdims=True)
            cand = jnp.where(dist == m, iota_k, K)
            idx = jnp.min(cand, axis=1)
            idx_rows[h].append(idx)
            onehot = (iota_k == idx[:, None]).astype(jnp.bfloat16)
            d3 = jax.lax.dot_general(
                onehot, cbs_ref[q], (((1,), (0,)), ((), ())),
                preferred_element_type=jnp.float32)
            qv = (d3[:, :D] + d3[:, D:2 * D]) + d3[:, 2 * D:]
            diff = qv - res[h]
            loss_q.append(jnp.sum(diff * diff, axis=0))
            res[h] = res[h] - qv
            qacc[h] = qacc[h] + qv
        acc = loss_q[0]
        for h in range(1, NCHAIN):
            acc = acc + loss_q[h]
        loss_rows.append(acc)
    qout_ref[...] = jnp.concatenate(qacc, axis=0)
    idx_ref[...] = jnp.concatenate(
        [jnp.stack(rows) for rows in idx_rows], axis=1)
    loss_ref[...] = jnp.stack(loss_rows)[None]


def kernel(x, codebooks):
    b, t, d = x.shape
    n = b * t
    tile = 2048
    nt = n // tile
    xf = x.reshape(n, d)
    mask = jnp.uint32(0xFFFF0000)
    u = jax.lax.bitcast_convert_type(codebooks, jnp.uint32)
    hi32 = jax.lax.bitcast_convert_type(u & mask, jnp.float32)
    rem = codebooks - hi32
    u2 = jax.lax.bitcast_convert_type(rem, jnp.uint32)
    mid32 = jax.lax.bitcast_convert_type(u2 & mask, jnp.float32)
    lo32 = rem - mid32
    cb_split = jnp.concatenate(
        [hi32.astype(jnp.bfloat16), mid32.astype(jnp.bfloat16),
         lo32.astype(jnp.bfloat16)], axis=-1)
    qout, idx, loss_parts = pl.pallas_call(
        _rvq_body,
        grid=(nt,),
        in_specs=[
            pl.BlockSpec((tile, d), lambda i: (i, 0)),
            pl.BlockSpec((Q, K, D), lambda i: (0, 0, 0)),
            pl.BlockSpec((Q, K, 3 * D), lambda i: (0, 0, 0)),
        ],
        out_specs=[
            pl.BlockSpec((tile, d), lambda i: (i, 0)),
            pl.BlockSpec((Q, tile), lambda i: (0, i)),
            pl.BlockSpec((1, Q, D), lambda i: (i, 0, 0)),
        ],
        out_shape=[
            jax.ShapeDtypeStruct((n, d), jnp.float32),
            jax.ShapeDtypeStruct((Q, n), jnp.int32),
            jax.ShapeDtypeStruct((nt, Q, D), jnp.float32),
        ],
        compiler_params=pltpu.CompilerParams(
            dimension_semantics=("parallel",),
        ),
    )(xf, codebooks, cb_split)
    sums = jnp.sum(loss_parts, axis=(0, 2))
    per_layer = sums / (n * d)
    out_loss = jnp.mean(per_layer + BETA * per_layer)
    return qout.reshape(b, t, d), idx.reshape(Q, b, t), out_loss


# speedup vs baseline: 1.4609x; 1.4609x over previous
import jax
import jax.numpy as jnp
from jax.experimental import pallas as pl
from jax.experimental.pallas import tpu as pltpu

Q = 8
K = 1024
D = 256
BETA = 0.25


NCHAIN = 2


def _rvq_body(x_ref, cb_ref, cbs_ref, qout_ref, idx_ref, loss_ref):
    r = x_ref.shape[0] // NCHAIN
    iota_k = jax.lax.broadcasted_iota(jnp.int32, (r, K), 1)
    res = [x_ref[h * r:(h + 1) * r] for h in range(NCHAIN)]
    qacc = [jnp.zeros_like(res[0]) for _ in range(NCHAIN)]
    idx_rows = [[] for _ in range(NCHAIN)]
    loss_rows = []
    for q in range(Q):
        cb = cb_ref[q]
        cnorm = jnp.sum(cb * cb, axis=1)
        loss_q = []
        for h in range(NCHAIN):
            mm = jax.lax.dot_general(
                res[h], cb, (((1,), (1,)), ((), ())),
                preferred_element_type=jnp.float32)
            rown = jnp.sum(res[h] * res[h], axis=1, keepdims=True)
            dist = rown - 2.0 * mm + cnorm[None, :]
            m = jnp.min(dist, axis=1, keepdims=True)
            cand = jnp.where(dist == m, iota_k, K)
            idx = jnp.min(cand, axis=1)
            idx_rows[h].append(idx)
            onehot = (iota_k == idx[:, None]).astype(jnp.bfloat16)
            d2 = jax.lax.dot_general(
                onehot, cbs_ref[q], (((1,), (0,)), ((), ())),
                preferred_element_type=jnp.float32)
            qv = d2[:, :D] + d2[:, D:]
            loss_q.append(jnp.sum(m))
            res[h] = res[h] - qv
            qacc[h] = qacc[h] + qv
        acc = loss_q[0]
        for h in range(1, NCHAIN):
            acc = acc + loss_q[h]
        loss_rows.append(jnp.full((D,), acc, dtype=jnp.float32))
    qout_ref[...] = jnp.concatenate(qacc, axis=0)
    idx_ref[...] = jnp.concatenate(
        [jnp.stack(rows) for rows in idx_rows], axis=1)
    loss_ref[...] = jnp.stack(loss_rows)[None]


def kernel(x, codebooks):
    b, t, d = x.shape
    n = b * t
    tile = 1024
    nt = n // tile
    xf = x.reshape(n, d)
    mask = jnp.uint32(0xFFFF0000)
    u = jax.lax.bitcast_convert_type(codebooks, jnp.uint32)
    hi32 = jax.lax.bitcast_convert_type(u & mask, jnp.float32)
    rem = codebooks - hi32
    u2 = jax.lax.bitcast_convert_type(rem, jnp.uint32)
    mid32 = jax.lax.bitcast_convert_type(u2 & mask, jnp.float32)
    cb_split = jnp.concatenate(
        [hi32.astype(jnp.bfloat16), mid32.astype(jnp.bfloat16)],
        axis=-1)
    qout, idx, loss_parts = pl.pallas_call(
        _rvq_body,
        grid=(nt,),
        in_specs=[
            pl.BlockSpec((tile, d), lambda i: (i, 0)),
            pl.BlockSpec((Q, K, D), lambda i: (0, 0, 0)),
            pl.BlockSpec((Q, K, 2 * D), lambda i: (0, 0, 0)),
        ],
        out_specs=[
            pl.BlockSpec((tile, d), lambda i: (i, 0)),
            pl.BlockSpec((Q, tile), lambda i: (0, i)),
            pl.BlockSpec((1, Q, D), lambda i: (i, 0, 0)),
        ],
        out_shape=[
            jax.ShapeDtypeStruct((n, d), jnp.float32),
            jax.ShapeDtypeStruct((Q, n), jnp.int32),
            jax.ShapeDtypeStruct((nt, Q, D), jnp.float32),
        ],
        compiler_params=pltpu.CompilerParams(
            dimension_semantics=("parallel",),
        ),
    )(xf, codebooks, cb_split)
    sums = jnp.sum(loss_parts[:, :, 0], axis=0)
    per_layer = sums / (n * d)
    out_loss = jnp.mean(per_layer + BETA * per_layer)
    return qout.reshape(b, t, d), idx.reshape(Q, b, t), out_loss
